# baseline (speedup 1.0000x reference)
import jax
import jax.numpy as jnp
from jax import lax
from jax.experimental import pallas as pl
from jax.experimental.pallas import tpu as pltpu
from jax.experimental.pallas import tpu_sc as plsc

NUM = 1000
W = 16
H = 16
L = 8
C = 12
CP = 16
T = 1048576

NC = 2
NS = 16
NW = NC * NS
PW = T // NW
K = 128
KSUB = 128
NSUB = K // KSUB
NCHUNK = PW // K
NG = K // 16
NSLOT = 4
ROW_PAD = 288
TROWS = NUM * L * H * W + ROW_PAD

DK = (0, 1, W, W + 1, H * W, H * W + 1, H * W + W, H * W + W + 1)


def _body(table, xyf, rgbf, idx_h, w16, out_h, wv, *slotrefs):
    wid = lax.axis_index("s") * NC + lax.axis_index("c")
    base_w = wid * PW

    pltpu.sync_copy(w16, wv)
    wvv = wv[...]
    w0 = wvv[0]
    w1 = wvv[1]
    w2 = wvv[2]

    slots = tuple(slotrefs[i * 10:(i + 1) * 10] for i in range(NSLOT))

    def issue_in(c, s):
        xyb, rgbb, ib, idxb, wtb, rows, ob, sin, sg, so = slots[s]
        p = base_w + c * K
        pltpu.async_copy(xyf.at[pl.ds(p, K)], xyb.at[pl.ds(0, K)], sin)
        pltpu.async_copy(xyf.at[pl.ds(T + p, K)], xyb.at[pl.ds(K, K)], sin)
        pltpu.async_copy(rgbf.at[pl.ds(p, K)], rgbb.at[pl.ds(0, K)], sin)
        pltpu.async_copy(rgbf.at[pl.ds(T + p, K)], rgbb.at[pl.ds(K, K)], sin)
        pltpu.async_copy(rgbf.at[pl.ds(2 * T + p, K)], rgbb.at[pl.ds(2 * K, K)], sin)
        pltpu.async_copy(idx_h.at[pl.ds(p, K)], ib, sin)

    def wait_in(c, s):
        xyb, rgbb, ib, idxb, wtb, rows, ob, sin, sg, so = slots[s]
        p = base_w + c * K
        pltpu.make_async_copy(xyf.at[pl.ds(p, K)], xyb.at[pl.ds(0, K)], sin).wait()
        pltpu.make_async_copy(xyf.at[pl.ds(T + p, K)], xyb.at[pl.ds(K, K)], sin).wait()
        pltpu.make_async_copy(rgbf.at[pl.ds(p, K)], rgbb.at[pl.ds(0, K)], sin).wait()
        pltpu.make_async_copy(rgbf.at[pl.ds(T + p, K)], rgbb.at[pl.ds(K, K)], sin).wait()
        pltpu.make_async_copy(rgbf.at[pl.ds(2 * T + p, K)], rgbb.at[pl.ds(2 * K, K)], sin).wait()
        pltpu.make_async_copy(idx_h.at[pl.ds(p, K)], ib, sin).wait()

    def coord(s):
        xyb, rgbb, ib, idxb, wtb, rows, ob, sin, sg, so = slots[s]

        def coord_g(j, g2, _):
            sl = j * KSUB + g2 * 16
            t = lax.iota(jnp.int32, 16) + sl
            xv = xyb[pl.ds(sl, 16)]
            yv = xyb[pl.ds(K + sl, 16)]
            rv = rgbb[pl.ds(sl, 16)]
            gv = rgbb[pl.ds(K + sl, 16)]
            bv = rgbb[pl.ds(2 * K + sl, 16)]
            iv = ib[pl.ds(sl, 16)]
            zraw = (rv * w0 + gv * w1 + bv * w2) * 2.0 - 1.0
            x2 = xv * 2.0 - 1.0
            y2 = yv * 2.0 - 1.0
            ix = jnp.clip((x2 + 1.0) * (0.5 * (W - 1)), 0.0, W - 1)
            iy = jnp.clip((y2 + 1.0) * (0.5 * (H - 1)), 0.0, H - 1)
            iz = jnp.clip((zraw + 1.0) * (0.5 * (L - 1)), 0.0, L - 1)
            x0 = ix.astype(jnp.int32)
            y0 = iy.astype(jnp.int32)
            z0 = iz.astype(jnp.int32)
            wx = ix - x0.astype(jnp.float32)
            wy = iy - y0.astype(jnp.float32)
            wz = iz - z0.astype(jnp.float32)
            ux = 1.0 - wx
            uy = 1.0 - wy
            uz = 1.0 - wz
            lin = iv * (L * H * W) + z0 * (H * W) + y0 * W + x0
            for k in range(8):
                idxb[k * NSUB + j, pl.ds(g2 * 16, 16)] = lin + DK[k]
            zy00 = uz * uy
            zy01 = uz * wy
            zy10 = wz * uy
            zy11 = wz * wy
            wtb[0, pl.ds(sl, 16)] = zy00 * ux
            wtb[1, pl.ds(sl, 16)] = zy00 * wx
            wtb[2, pl.ds(sl, 16)] = zy01 * ux
            wtb[3, pl.ds(sl, 16)] = zy01 * wx
            wtb[4, pl.ds(sl, 16)] = zy10 * ux
            wtb[5, pl.ds(sl, 16)] = zy10 * wx
            wtb[6, pl.ds(sl, 16)] = zy11 * ux
            wtb[7, pl.ds(sl, 16)] = zy11 * wx
            wtb[8, pl.ds(sl, 16)] = rv
            wtb[9, pl.ds(sl, 16)] = gv
            wtb[10, pl.ds(sl, 16)] = bv
            return 0

        for j in range(NSUB):
            lax.fori_loop(0, KSUB // 16,
                          lambda g2, c, j=j: coord_g(j, g2, c), 0,
                          unroll=False)

    def issue_gathers(s):
        xyb, rgbb, ib, idxb, wtb, rows, ob, sin, sg, so = slots[s]
        for k in range(8):
            for j in range(NSUB):
                pltpu.async_copy(
                    table.at[idxb.at[k * NSUB + j]],
                    rows.at[k, pl.ds(j * KSUB, KSUB)], sg)

    def wait_gathers(s):
        xyb, rgbb, ib, idxb, wtb, rows, ob, sin, sg, so = slots[s]
        for k in range(8):
            for j in range(NSUB):
                pltpu.make_async_copy(
                    table.at[idxb.at[k * NSUB + j]],
                    rows.at[k, pl.ds(j * KSUB, KSUB)], sg).wait()

    def blend(s):
        xyb, rgbb, ib, idxb, wtb, rows, ob, sin, sg, so = slots[s]

        def blend_g(g, _):
            sl = g * 16
            t = lax.iota(jnp.int32, 16) + sl
            rv = wtb[8, pl.ds(sl, 16)]
            gv = wtb[9, pl.ds(sl, 16)]
            bv = wtb[10, pl.ds(sl, 16)]
            wk = [wtb[k, pl.ds(sl, 16)] for k in range(8)]
            v = []
            for ch in range(C):
                cs = jnp.full((16,), ch, jnp.int32)
                acc = None
                for k in range(8):
                    ks = jnp.full((16,), k, jnp.int32)
                    g8 = plsc.load_gather(rows, [ks, t, cs])
                    acc = g8 * wk[k] if acc is None else acc + g8 * wk[k]
                v.append(acc)
            o0 = v[0] * rv + v[1] * gv + v[2] * bv + v[3]
            o1 = v[4] * rv + v[5] * gv + v[6] * bv + v[7]
            o2 = v[8] * rv + v[9] * gv + v[10] * bv + v[11]
            ob[pl.ds(sl, 16)] = o0
            ob[pl.ds(K + sl, 16)] = o1
            ob[pl.ds(2 * K + sl, 16)] = o2
            return 0

        lax.fori_loop(0, NG, blend_g, 0, unroll=False)

    def issue_out(c, s):
        xyb, rgbb, ib, idxb, wtb, rows, ob, sin, sg, so = slots[s]
        p = base_w + c * K
        pltpu.async_copy(ob.at[pl.ds(0, K)], out_h.at[pl.ds(p, K)], so)
        pltpu.async_copy(ob.at[pl.ds(K, K)], out_h.at[pl.ds(T + p, K)], so)
        pltpu.async_copy(ob.at[pl.ds(2 * K, K)], out_h.at[pl.ds(2 * T + p, K)], so)

    def wait_out(c, s):
        xyb, rgbb, ib, idxb, wtb, rows, ob, sin, sg, so = slots[s]
        p = base_w + c * K
        pltpu.make_async_copy(ob.at[pl.ds(0, K)], out_h.at[pl.ds(p, K)], so).wait()
        pltpu.make_async_copy(ob.at[pl.ds(K, K)], out_h.at[pl.ds(T + p, K)], so).wait()
        pltpu.make_async_copy(ob.at[pl.ds(2 * K, K)], out_h.at[pl.ds(2 * T + p, K)], so).wait()

    issue_in(0, 0)
    issue_in(1, 1)
    wait_in(0, 0)
    coord(0)
    issue_gathers(0)
    issue_in(2, 2)
    wait_in(1, 1)
    coord(1)
    issue_gathers(1)
    issue_in(3, 3)

    def stage(c, cc, u):
        s = (2 + u) % NSLOT
        sb = u % NSLOT
        wait_in(c, s)
        coord(s)
        issue_gathers(s)
        issue_in(c + 2, sb)

        @pl.when(cc > 0)
        def _():
            wait_out(c - 6, sb)

        wait_gathers(sb)
        blend(sb)
        issue_out(c - 2, sb)

    def step(cc, _):
        for u in range(4):
            c = 4 * cc + 2 + u
            stage(c, cc, u)
        return 0

    lax.fori_loop(0, (NCHUNK - 2) // 4, step, 0, unroll=False)

    for c in (NCHUNK - 2, NCHUNK - 1):
        s = c % NSLOT
        sb = (c - 2) % NSLOT
        wait_in(c, s)
        coord(s)
        issue_gathers(s)
        wait_out(c - 6, sb)
        wait_gathers(sb)
        blend(sb)
        issue_out(c - 2, sb)
    for c in (NCHUNK - 2, NCHUNK - 1):
        s = c % NSLOT
        wait_out(c - 4, s)
        wait_gathers(s)
        blend(s)
        issue_out(c, s)
    for c in range(NCHUNK - 4, NCHUNK):
        wait_out(c, c % NSLOT)


GRID_CELLS = L * H * W
GRID_FLOATS = C * GRID_CELLS
GPW = (NUM + NW - 1) // NW


def _table_body(g1d, table, gin, gout, sem):
    wid = lax.axis_index("s") * NC + lax.axis_index("c")

    zv = jnp.zeros((16,), jnp.float32)

    @pl.when(wid == 0)
    def _():
        def zg(g, _):
            gout[pl.ds(g * 16, 16)] = zv
            return 0
        lax.fori_loop(0, ROW_PAD, zg, 0, unroll=False)
        pltpu.async_copy(
            gout.at[pl.ds(0, ROW_PAD * CP)],
            table.at[pl.ds(NUM * GRID_CELLS * CP, ROW_PAD * CP)], sem).wait()

    def per_grid(j, _):
        n = j * NW + wid

        @pl.when(n < NUM)
        def _():
            pltpu.async_copy(
                g1d.at[pl.ds(n * GRID_FLOATS, GRID_FLOATS)], gin, sem).wait()

            def tr_g(g, _):
                t = lax.iota(jnp.int32, 16) + g * 16
                tg = t * CP
                for ch in range(C):
                    v = plsc.load_gather(gin, [t + ch * GRID_CELLS])
                    plsc.store_scatter(gout, [tg + ch], v)
                return 0

            lax.fori_loop(0, GRID_CELLS // 16, tr_g, 0, unroll=False)
            pltpu.async_copy(
                gout,
                table.at[pl.ds(n * GRID_CELLS * CP, GRID_CELLS * CP)],
                sem).wait()

        return 0

    lax.fori_loop(0, GPW, per_grid, 0, unroll=False)


def _make_table(grids):
    mesh = plsc.VectorSubcoreMesh(
        core_axis_name="c", subcore_axis_name="s",
        num_cores=NC, num_subcores=NS)
    run = pl.kernel(
        _table_body,
        out_type=jax.ShapeDtypeStruct((TROWS * CP,), jnp.float32),
        mesh=mesh,
        scratch_types=[
            pltpu.VMEM((GRID_FLOATS,), jnp.float32),
            pltpu.VMEM((GRID_CELLS * CP,), jnp.float32),
            pltpu.SemaphoreType.DMA,
        ],
        compiler_params=pltpu.CompilerParams(
            needs_layout_passes=False, use_tc_tiling_on_sc=False),
    )
    return run(grids.reshape(-1)).reshape(TROWS, CP)


def kernel(grids, rgb2gray_weight, grid_xy, rgb, idx):
    table = _make_table(grids)
    xyf = grid_xy.T.reshape(-1)
    rgbf = rgb.T.reshape(-1)
    w16 = jnp.pad(rgb2gray_weight.reshape(-1), (0, 13))

    mesh = plsc.VectorSubcoreMesh(
        core_axis_name="c", subcore_axis_name="s",
        num_cores=NC, num_subcores=NS)
    run = pl.kernel(
        _body,
        out_type=jax.ShapeDtypeStruct((3 * T,), jnp.float32),
        mesh=mesh,
        scratch_types=[pltpu.VMEM((16,), jnp.float32)] + [
            st
            for _ in range(NSLOT)
            for st in (
                pltpu.VMEM((2 * K,), jnp.float32),
                pltpu.VMEM((3 * K,), jnp.float32),
                pltpu.VMEM((K,), jnp.int32),
                pltpu.VMEM((8 * NSUB, KSUB), jnp.int32),
                pltpu.VMEM((11, K), jnp.float32),
                pltpu.VMEM((8, K, CP), jnp.float32),
                pltpu.VMEM((3 * K,), jnp.float32),
                pltpu.SemaphoreType.DMA,
                pltpu.SemaphoreType.DMA,
                pltpu.SemaphoreType.DMA,
            )
        ],
        compiler_params=pltpu.CompilerParams(
            needs_layout_passes=False, use_tc_tiling_on_sc=False),
    )
    out3 = run(table, xyf, rgbf, idx, w16)
    return out3.reshape(3, T).T


kernel = jax.jit(kernel)


# speedup vs baseline: 1.0629x; 1.0629x over previous
import jax
import jax.numpy as jnp
from jax import lax
from jax.experimental import pallas as pl
from jax.experimental.pallas import tpu as pltpu
from jax.experimental.pallas import tpu_sc as plsc

NUM = 1000
W = 16
H = 16
L = 8
C = 12
CP = 16
T = 1048576

NC = 2
NS = 16
NW = NC * NS
PW = T // NW
K = 128
KSUB = 128
NSUB = K // KSUB
NCHUNK = PW // K
NG = K // 16
NSLOT = 4
ROW_PAD = 288
TROWS = NUM * L * H * W + ROW_PAD

DK = (0, 1, W, W + 1, H * W, H * W + 1, H * W + W, H * W + W + 1)


def _body(table, xyf, rgbf, idx_h, w16, out_h, wv, *slotrefs):
    wid = lax.axis_index("s") * NC + lax.axis_index("c")
    base_w = wid * PW

    pltpu.sync_copy(w16, wv)
    wvv = wv[...]
    w0 = wvv[0]
    w1 = wvv[1]
    w2 = wvv[2]

    slots = tuple(slotrefs[i * 10:(i + 1) * 10] for i in range(NSLOT))

    def issue_in(c, s):
        xyb, rgbb, ib, idxb, wtb, rows, ob, sin, sg, so = slots[s]
        p = base_w + c * K
        pltpu.async_copy(xyf.at[pl.ds(p, K)], xyb.at[pl.ds(0, K)], sin)
        pltpu.async_copy(xyf.at[pl.ds(T + p, K)], xyb.at[pl.ds(K, K)], sin)
        pltpu.async_copy(rgbf.at[pl.ds(p, K)], rgbb.at[pl.ds(0, K)], sin)
        pltpu.async_copy(rgbf.at[pl.ds(T + p, K)], rgbb.at[pl.ds(K, K)], sin)
        pltpu.async_copy(rgbf.at[pl.ds(2 * T + p, K)], rgbb.at[pl.ds(2 * K, K)], sin)
        pltpu.async_copy(idx_h.at[pl.ds(p, K)], ib, sin)

    def wait_in(c, s):
        xyb, rgbb, ib, idxb, wtb, rows, ob, sin, sg, so = slots[s]
        p = base_w + c * K
        pltpu.make_async_copy(xyf.at[pl.ds(p, K)], xyb.at[pl.ds(0, K)], sin).wait()
        pltpu.make_async_copy(xyf.at[pl.ds(T + p, K)], xyb.at[pl.ds(K, K)], sin).wait()
        pltpu.make_async_copy(rgbf.at[pl.ds(p, K)], rgbb.at[pl.ds(0, K)], sin).wait()
        pltpu.make_async_copy(rgbf.at[pl.ds(T + p, K)], rgbb.at[pl.ds(K, K)], sin).wait()
        pltpu.make_async_copy(rgbf.at[pl.ds(2 * T + p, K)], rgbb.at[pl.ds(2 * K, K)], sin).wait()
        pltpu.make_async_copy(idx_h.at[pl.ds(p, K)], ib, sin).wait()

    def coord(s):
        xyb, rgbb, ib, idxb, wtb, rows, ob, sin, sg, so = slots[s]

        def coord_g(j, g2, _):
            sl = j * KSUB + g2 * 16
            t = lax.iota(jnp.int32, 16) + sl
            xv = xyb[pl.ds(sl, 16)]
            yv = xyb[pl.ds(K + sl, 16)]
            rv = rgbb[pl.ds(sl, 16)]
            gv = rgbb[pl.ds(K + sl, 16)]
            bv = rgbb[pl.ds(2 * K + sl, 16)]
            iv = ib[pl.ds(sl, 16)]
            zraw = (rv * w0 + gv * w1 + bv * w2) * 2.0 - 1.0
            x2 = xv * 2.0 - 1.0
            y2 = yv * 2.0 - 1.0
            ix = jnp.clip((x2 + 1.0) * (0.5 * (W - 1)), 0.0, W - 1)
            iy = jnp.clip((y2 + 1.0) * (0.5 * (H - 1)), 0.0, H - 1)
            iz = jnp.clip((zraw + 1.0) * (0.5 * (L - 1)), 0.0, L - 1)
            x0 = ix.astype(jnp.int32)
            y0 = iy.astype(jnp.int32)
            z0 = iz.astype(jnp.int32)
            wx = ix - x0.astype(jnp.float32)
            wy = iy - y0.astype(jnp.float32)
            wz = iz - z0.astype(jnp.float32)
            ux = 1.0 - wx
            uy = 1.0 - wy
            uz = 1.0 - wz
            lin = iv * (L * H * W) + z0 * (H * W) + y0 * W + x0
            for k in range(8):
                idxb[k * NSUB + j, pl.ds(g2 * 16, 16)] = lin + DK[k]
            zy00 = uz * uy
            zy01 = uz * wy
            zy10 = wz * uy
            zy11 = wz * wy
            wtb[0, pl.ds(sl, 16)] = zy00 * ux
            wtb[1, pl.ds(sl, 16)] = zy00 * wx
            wtb[2, pl.ds(sl, 16)] = zy01 * ux
            wtb[3, pl.ds(sl, 16)] = zy01 * wx
            wtb[4, pl.ds(sl, 16)] = zy10 * ux
            wtb[5, pl.ds(sl, 16)] = zy10 * wx
            wtb[6, pl.ds(sl, 16)] = zy11 * ux
            wtb[7, pl.ds(sl, 16)] = zy11 * wx
            wtb[8, pl.ds(sl, 16)] = rv
            wtb[9, pl.ds(sl, 16)] = gv
            wtb[10, pl.ds(sl, 16)] = bv
            return 0

        for j in range(NSUB):
            lax.fori_loop(0, KSUB // 16,
                          lambda g2, c, j=j: coord_g(j, g2, c), 0,
                          unroll=False)

    def issue_gathers(s):
        xyb, rgbb, ib, idxb, wtb, rows, ob, sin, sg, so = slots[s]
        for k in range(8):
            for j in range(NSUB):
                pltpu.async_copy(
                    table.at[idxb.at[k * NSUB + j]],
                    rows.at[k, pl.ds(j * KSUB, KSUB)], sg)

    def wait_gathers(s):
        xyb, rgbb, ib, idxb, wtb, rows, ob, sin, sg, so = slots[s]
        for k in range(8):
            for j in range(NSUB):
                pltpu.make_async_copy(
                    table.at[idxb.at[k * NSUB + j]],
                    rows.at[k, pl.ds(j * KSUB, KSUB)], sg).wait()

    def blend(s):
        xyb, rgbb, ib, idxb, wtb, rows, ob, sin, sg, so = slots[s]

        def blend_g(g, _):
            sl = g * 16
            t = lax.iota(jnp.int32, 16) + sl
            rv = wtb[8, pl.ds(sl, 16)]
            gv = wtb[9, pl.ds(sl, 16)]
            bv = wtb[10, pl.ds(sl, 16)]
            wk = [wtb[k, pl.ds(sl, 16)] for k in range(8)]
            v = []
            for ch in range(C):
                cs = jnp.full((16,), ch, jnp.int32)
                acc = None
                for k in range(8):
                    ks = jnp.full((16,), k, jnp.int32)
                    g8 = plsc.load_gather(rows, [ks, t, cs])
                    acc = g8 * wk[k] if acc is None else acc + g8 * wk[k]
                v.append(acc)
            o0 = v[0] * rv + v[1] * gv + v[2] * bv + v[3]
            o1 = v[4] * rv + v[5] * gv + v[6] * bv + v[7]
            o2 = v[8] * rv + v[9] * gv + v[10] * bv + v[11]
            ob[pl.ds(sl, 16)] = o0
            ob[pl.ds(K + sl, 16)] = o1
            ob[pl.ds(2 * K + sl, 16)] = o2
            return 0

        lax.fori_loop(0, NG, blend_g, 0, unroll=False)

    def issue_out(c, s):
        xyb, rgbb, ib, idxb, wtb, rows, ob, sin, sg, so = slots[s]
        p = base_w + c * K
        pltpu.async_copy(ob.at[pl.ds(0, K)], out_h.at[pl.ds(p, K)], so)
        pltpu.async_copy(ob.at[pl.ds(K, K)], out_h.at[pl.ds(T + p, K)], so)
        pltpu.async_copy(ob.at[pl.ds(2 * K, K)], out_h.at[pl.ds(2 * T + p, K)], so)

    def wait_out(c, s):
        xyb, rgbb, ib, idxb, wtb, rows, ob, sin, sg, so = slots[s]
        p = base_w + c * K
        pltpu.make_async_copy(ob.at[pl.ds(0, K)], out_h.at[pl.ds(p, K)], so).wait()
        pltpu.make_async_copy(ob.at[pl.ds(K, K)], out_h.at[pl.ds(T + p, K)], so).wait()
        pltpu.make_async_copy(ob.at[pl.ds(2 * K, K)], out_h.at[pl.ds(2 * T + p, K)], so).wait()

    issue_in(0, 0)
    issue_in(1, 1)
    wait_in(0, 0)
    coord(0)
    issue_gathers(0)
    issue_in(2, 2)
    wait_in(1, 1)
    coord(1)
    issue_gathers(1)
    issue_in(3, 3)

    def stage(c, cc, u):
        s = (2 + u) % NSLOT
        sb = u % NSLOT
        wait_in(c, s)
        coord(s)
        issue_gathers(s)
        issue_in(c + 2, sb)

        @pl.when(cc > 0)
        def _():
            wait_out(c - 6, sb)

        wait_gathers(sb)
        blend(sb)
        issue_out(c - 2, sb)

    def step(cc, _):
        for u in range(4):
            c = 4 * cc + 2 + u
            stage(c, cc, u)
        return 0

    lax.fori_loop(0, (NCHUNK - 2) // 4, step, 0, unroll=False)

    for c in (NCHUNK - 2, NCHUNK - 1):
        s = c % NSLOT
        sb = (c - 2) % NSLOT
        wait_in(c, s)
        coord(s)
        issue_gathers(s)
        wait_out(c - 6, sb)
        wait_gathers(sb)
        blend(sb)
        issue_out(c - 2, sb)
    for c in (NCHUNK - 2, NCHUNK - 1):
        s = c % NSLOT
        wait_out(c - 4, s)
        wait_gathers(s)
        blend(s)
        issue_out(c, s)
    for c in range(NCHUNK - 4, NCHUNK):
        wait_out(c, c % NSLOT)


GRID_CELLS = L * H * W
GRID_FLOATS = C * GRID_CELLS
GPW = (NUM + NW - 1) // NW


def _table_body(g1d, table, gin0, gout0, gin1, gout1, si0, si1, so0, so1):
    wid = lax.axis_index("s") * NC + lax.axis_index("c")

    zv = jnp.zeros((16,), jnp.float32)

    @pl.when(wid == 0)
    def _():
        def zg(g, _):
            gout0[pl.ds(g * 16, 16)] = zv
            return 0
        lax.fori_loop(0, ROW_PAD, zg, 0, unroll=False)
        pltpu.async_copy(
            gout0.at[pl.ds(0, ROW_PAD * CP)],
            table.at[pl.ds(NUM * GRID_CELLS * CP, ROW_PAD * CP)], si0).wait()

    slots = ((gin0, gout0, si0, so0), (gin1, gout1, si1, so1))

    def issue_in(j, s):
        gin, gout, si, so = slots[s]
        n = j * NW + wid
        pltpu.async_copy(g1d.at[pl.ds(n * GRID_FLOATS, GRID_FLOATS)], gin, si)

    def wait_in(j, s):
        gin, gout, si, so = slots[s]
        n = j * NW + wid
        pltpu.make_async_copy(
            g1d.at[pl.ds(n * GRID_FLOATS, GRID_FLOATS)], gin, si).wait()

    def transpose(s):
        gin, gout, si, so = slots[s]

        def tr_g(g, _):
            t = lax.iota(jnp.int32, 16) + g * 16
            tg = t * CP
            for ch in range(C):
                v = plsc.load_gather(gin, [t + ch * GRID_CELLS])
                plsc.store_scatter(gout, [tg + ch], v)
            return 0

        lax.fori_loop(0, GRID_CELLS // 16, tr_g, 0, unroll=False)

    def issue_out(j, s):
        gin, gout, si, so = slots[s]
        n = j * NW + wid
        pltpu.async_copy(
            gout, table.at[pl.ds(n * GRID_CELLS * CP, GRID_CELLS * CP)], so)

    def wait_out(j, s):
        gin, gout, si, so = slots[s]
        n = j * NW + wid
        pltpu.make_async_copy(
            gout, table.at[pl.ds(n * GRID_CELLS * CP, GRID_CELLS * CP)],
            so).wait()

    def valid(j):
        return (j * NW + wid) < NUM

    @pl.when(valid(0))
    def _():
        issue_in(0, 0)

    def per_pair(jj, _):
        for u in range(2):
            j = 2 * jj + u

            @pl.when(valid(j))
            def _(j=j, u=u):
                wait_in(j, u)

                @pl.when(valid(j + 1))
                def _():
                    issue_in(j + 1, 1 - u)

                @pl.when((jj > 0) & valid(j - 2))
                def _():
                    wait_out(j - 2, u)

                transpose(u)
                issue_out(j, u)

        return 0

    lax.fori_loop(0, GPW // 2, per_pair, 0, unroll=False)

    for j in (GPW - 2, GPW - 1):
        @pl.when(valid(j))
        def _(j=j):
            wait_out(j, j % 2)


def _make_table(grids):
    mesh = plsc.VectorSubcoreMesh(
        core_axis_name="c", subcore_axis_name="s",
        num_cores=NC, num_subcores=NS)
    run = pl.kernel(
        _table_body,
        out_type=jax.ShapeDtypeStruct((TROWS * CP,), jnp.float32),
        mesh=mesh,
        scratch_types=[
            pltpu.VMEM((GRID_FLOATS,), jnp.float32),
            pltpu.VMEM((GRID_CELLS * CP,), jnp.float32),
            pltpu.VMEM((GRID_FLOATS,), jnp.float32),
            pltpu.VMEM((GRID_CELLS * CP,), jnp.float32),
            pltpu.SemaphoreType.DMA,
            pltpu.SemaphoreType.DMA,
            pltpu.SemaphoreType.DMA,
            pltpu.SemaphoreType.DMA,
        ],
        compiler_params=pltpu.CompilerParams(
            needs_layout_passes=False, use_tc_tiling_on_sc=False),
    )
    return run(grids.reshape(-1)).reshape(TROWS, CP)


def kernel(grids, rgb2gray_weight, grid_xy, rgb, idx):
    table = _make_table(grids)
    xyf = grid_xy.T.reshape(-1)
    rgbf = rgb.T.reshape(-1)
    w16 = jnp.pad(rgb2gray_weight.reshape(-1), (0, 13))

    mesh = plsc.VectorSubcoreMesh(
        core_axis_name="c", subcore_axis_name="s",
        num_cores=NC, num_subcores=NS)
    run = pl.kernel(
        _body,
        out_type=jax.ShapeDtypeStruct((3 * T,), jnp.float32),
        mesh=mesh,
        scratch_types=[pltpu.VMEM((16,), jnp.float32)] + [
            st
            for _ in range(NSLOT)
            for st in (
                pltpu.VMEM((2 * K,), jnp.float32),
                pltpu.VMEM((3 * K,), jnp.float32),
                pltpu.VMEM((K,), jnp.int32),
                pltpu.VMEM((8 * NSUB, KSUB), jnp.int32),
                pltpu.VMEM((11, K), jnp.float32),
                pltpu.VMEM((8, K, CP), jnp.float32),
                pltpu.VMEM((3 * K,), jnp.float32),
                pltpu.SemaphoreType.DMA,
                pltpu.SemaphoreType.DMA,
                pltpu.SemaphoreType.DMA,
            )
        ],
        compiler_params=pltpu.CompilerParams(
            needs_layout_passes=False, use_tc_tiling_on_sc=False),
    )
    out3 = run(table, xyf, rgbf, idx, w16)
    return out3.reshape(3, T).T


kernel = jax.jit(kernel)
